# Initial kernel scaffold; baseline (speedup 1.0000x reference)
import jax
import jax.numpy as jnp
from jax.experimental import pallas as pl


def kernel(x, a, e, kn_w1, kn_b1, root1, bias1, kn_w2, kn_b2, root2, bias2, dense_w, dense_b):
    raise NotImplementedError("write your pallas kernel here")


# speedup vs baseline: 27.3692x; 27.3692x over previous
import jax
import jax.numpy as jnp
from jax.experimental import pallas as pl
from jax.experimental.pallas import tpu as pltpu

B, N, F, S, C, NOUT = 32, 32, 32, 4, 32, 16


def _net_kernel(x2_ref, a2_ref, et_ref, w1_ref, wb1_ref, root1_ref, b1_ref,
                w2_ref, wb2_ref, root2_ref, b2_ref, dw_ref, db_ref, out_ref,
                t_s, u_s, r_s, h_s):
    f32 = jnp.float32
    feats = x2_ref[:, :F]

    t_s[:] = jnp.dot(feats, w1_ref[:], preferred_element_type=f32)
    u_s[:] = jnp.dot(feats, wb1_ref[:], preferred_element_type=f32)
    r_s[:] = jnp.dot(feats, root1_ref[:], preferred_element_type=f32) + b1_ref[:]

    def conv_rows(b):
        ab = a2_ref[pl.ds(b * N, N), :]
        a4 = jnp.concatenate([ab, ab, ab, ab], axis=1)
        ae = jnp.concatenate(
            [a4 * et_ref[pl.ds(b * N, N), :], ab], axis=1)
        tb = t_s[pl.ds(b * N, N), :]
        tu = jnp.concatenate(
            [tb[:, 0:C], tb[:, C:2 * C], tb[:, 2 * C:3 * C], tb[:, 3 * C:4 * C],
             u_s[pl.ds(b * N, N), :]], axis=0)
        rb = r_s[pl.ds(b * N, N), :]
        return jnp.maximum(jnp.dot(ae, tu, preferred_element_type=f32) + rb, 0.0)

    def body1(b, carry):
        h_s[pl.ds(b * N, N), :] = conv_rows(b)
        return carry

    jax.lax.fori_loop(0, B, body1, 0)

    h1 = h_s[:]
    t_s[:] = jnp.dot(h1, w2_ref[:], preferred_element_type=f32)
    u_s[:] = jnp.dot(h1, wb2_ref[:], preferred_element_type=f32)
    r_s[:] = jnp.dot(h1, root2_ref[:], preferred_element_type=f32) + b2_ref[:]

    def body2(b, carry):
        h2 = conv_rows(b)
        mb = (x2_ref[pl.ds(b * N, N), F:F + 1] != 0.0).astype(f32)
        pooled = jnp.sum(h2 * mb, axis=0, keepdims=True)
        out_ref[pl.ds(b, 1), :] = (
            jnp.dot(pooled, dw_ref[:], preferred_element_type=f32) + db_ref[:])
        return carry

    jax.lax.fori_loop(0, B, body2, 0)


def kernel(x, a, e, kn_w1, kn_b1, root1, bias1, kn_w2, kn_b2, root2, bias2,
           dense_w, dense_b):
    f32 = jnp.float32
    x2 = x.reshape(B * N, F + 1)
    a2 = a.reshape(B * N, N)
    et = e.transpose(0, 1, 3, 2).reshape(B * N, S * N)
    w1 = kn_w1.reshape(S, F, C).transpose(1, 0, 2).reshape(F, S * C)
    w2 = kn_w2.reshape(S, C, C).transpose(1, 0, 2).reshape(C, S * C)
    wb1 = kn_b1.reshape(F, C)
    wb2 = kn_b2.reshape(C, C)

    return pl.pallas_call(
        _net_kernel,
        out_shape=jax.ShapeDtypeStruct((B, NOUT), f32),
        scratch_shapes=[
            pltpu.VMEM((B * N, S * C), f32),
            pltpu.VMEM((B * N, C), f32),
            pltpu.VMEM((B * N, C), f32),
            pltpu.VMEM((B * N, C), f32),
        ],
    )(x2, a2, et, w1, wb1, root1, bias1.reshape(1, C),
      w2, wb2, root2, bias2.reshape(1, C), dense_w, dense_b.reshape(1, NOUT))


# speedup vs baseline: 55.1031x; 2.0133x over previous
import jax
import jax.numpy as jnp
from jax.experimental import pallas as pl
from jax.experimental.pallas import tpu as pltpu

B, N, F, S, C, NOUT = 32, 32, 32, 4, 32, 16


def _net_kernel(x2_ref, a2_ref, et_ref, w1_ref, wb1_ref, root1_ref, b1_ref,
                w2_ref, wb2_ref, root2_ref, b2_ref, dw_ref, db_ref, out_ref,
                ae_s, t_s, u_s, r_s, h_s):
    f32 = jnp.float32
    feats = x2_ref[:, :F]

    a2 = a2_ref[:]
    a4 = jnp.concatenate([a2, a2, a2, a2], axis=1)
    ae_s[:] = jnp.concatenate([a4 * et_ref[:], a2], axis=1)

    t_s[:] = jnp.dot(feats, w1_ref[:], preferred_element_type=f32)
    u_s[:] = jnp.dot(feats, wb1_ref[:], preferred_element_type=f32)
    r_s[:] = jnp.dot(feats, root1_ref[:], preferred_element_type=f32) + b1_ref[:]

    def conv_rows(b):
        ae = ae_s[b * N:(b + 1) * N, :]
        tb = t_s[b * N:(b + 1) * N, :]
        tu = jnp.concatenate(
            [tb[:, 0:C], tb[:, C:2 * C], tb[:, 2 * C:3 * C], tb[:, 3 * C:4 * C],
             u_s[b * N:(b + 1) * N, :]], axis=0)
        rb = r_s[b * N:(b + 1) * N, :]
        return jnp.maximum(jnp.dot(ae, tu, preferred_element_type=f32) + rb, 0.0)

    for b in range(B):
        h_s[b * N:(b + 1) * N, :] = conv_rows(b)

    h1 = h_s[:]
    t_s[:] = jnp.dot(h1, w2_ref[:], preferred_element_type=f32)
    u_s[:] = jnp.dot(h1, wb2_ref[:], preferred_element_type=f32)
    r_s[:] = jnp.dot(h1, root2_ref[:], preferred_element_type=f32) + b2_ref[:]

    mcol = (x2_ref[:, F:F + 1] != 0.0).astype(f32)
    rows = []
    for b in range(B):
        h2 = conv_rows(b)
        mb = mcol[b * N:(b + 1) * N, :]
        rows.append(jnp.sum(h2 * mb, axis=0, keepdims=True))
    pooled = jnp.concatenate(rows, axis=0)
    out_ref[:] = jnp.dot(pooled, dw_ref[:],
                         preferred_element_type=f32) + db_ref[:]


def kernel(x, a, e, kn_w1, kn_b1, root1, bias1, kn_w2, kn_b2, root2, bias2,
           dense_w, dense_b):
    f32 = jnp.float32
    x2 = x.reshape(B * N, F + 1)
    a2 = a.reshape(B * N, N)
    et = e.transpose(0, 1, 3, 2).reshape(B * N, S * N)
    w1 = kn_w1.reshape(S, F, C).transpose(1, 0, 2).reshape(F, S * C)
    w2 = kn_w2.reshape(S, C, C).transpose(1, 0, 2).reshape(C, S * C)
    wb1 = kn_b1.reshape(F, C)
    wb2 = kn_b2.reshape(C, C)

    return pl.pallas_call(
        _net_kernel,
        out_shape=jax.ShapeDtypeStruct((B, NOUT), f32),
        scratch_shapes=[
            pltpu.VMEM((B * N, S * N + N), f32),
            pltpu.VMEM((B * N, S * C), f32),
            pltpu.VMEM((B * N, C), f32),
            pltpu.VMEM((B * N, C), f32),
            pltpu.VMEM((B * N, C), f32),
        ],
    )(x2, a2, et, w1, wb1, root1, bias1.reshape(1, C),
      w2, wb2, root2, bias2.reshape(1, C), dense_w, dense_b.reshape(1, NOUT))


# speedup vs baseline: 61.4605x; 1.1154x over previous
import jax
import jax.numpy as jnp
import numpy as np
from jax.experimental import pallas as pl
from jax.experimental.pallas import tpu as pltpu

B, N, F, S, C, NOUT = 32, 32, 32, 4, 32, 16

_PERM = np.zeros((N * S, S * N), np.float32)
for _j in range(N):
    for _s in range(S):
        _PERM[_j * S + _s, _s * N + _j] = 1.0


def _net_kernel(x2_ref, a2_ref, e2_ref, p_ref, w1_ref, wb1_ref, root1_ref,
                b1_ref, w2_ref, wb2_ref, root2_ref, b2_ref, dw_ref, db_ref,
                out_ref, ae_s, t_s, u_s, r_s, h_s):
    f32 = jnp.float32
    feats = x2_ref[:, :F]

    et = jnp.dot(e2_ref[:], p_ref[:], preferred_element_type=f32)
    a2 = a2_ref[:]
    a4 = jnp.concatenate([a2, a2, a2, a2], axis=1)
    ae_s[:] = jnp.concatenate([a4 * et, a2], axis=1)

    t_s[:] = jnp.dot(feats, w1_ref[:], preferred_element_type=f32)
    u_s[:] = jnp.dot(feats, wb1_ref[:], preferred_element_type=f32)
    r_s[:] = jnp.dot(feats, root1_ref[:], preferred_element_type=f32) + b1_ref[:]

    def conv_rows(b):
        ae = ae_s[b * N:(b + 1) * N, :]
        tb = t_s[b * N:(b + 1) * N, :]
        tu = jnp.concatenate(
            [tb[:, 0:C], tb[:, C:2 * C], tb[:, 2 * C:3 * C], tb[:, 3 * C:4 * C],
             u_s[b * N:(b + 1) * N, :]], axis=0)
        rb = r_s[b * N:(b + 1) * N, :]
        return jnp.maximum(jnp.dot(ae, tu, preferred_element_type=f32) + rb, 0.0)

    for b in range(B):
        h_s[b * N:(b + 1) * N, :] = conv_rows(b)

    h1 = h_s[:]
    t_s[:] = jnp.dot(h1, w2_ref[:], preferred_element_type=f32)
    u_s[:] = jnp.dot(h1, wb2_ref[:], preferred_element_type=f32)
    r_s[:] = jnp.dot(h1, root2_ref[:], preferred_element_type=f32) + b2_ref[:]

    mcol = (x2_ref[:, F:F + 1] != 0.0).astype(f32)
    rows = []
    for b in range(B):
        h2 = conv_rows(b)
        mb = mcol[b * N:(b + 1) * N, :]
        rows.append(jnp.sum(h2 * mb, axis=0, keepdims=True))
    pooled = jnp.concatenate(rows, axis=0)
    out_ref[:] = jnp.dot(pooled, dw_ref[:],
                         preferred_element_type=f32) + db_ref[:]


def kernel(x, a, e, kn_w1, kn_b1, root1, bias1, kn_w2, kn_b2, root2, bias2,
           dense_w, dense_b):
    f32 = jnp.float32
    x2 = x.reshape(B * N, F + 1)
    a2 = a.reshape(B * N, N)
    e2 = e.reshape(B * N, N * S)
    w1 = kn_w1.reshape(S, F, C).transpose(1, 0, 2).reshape(F, S * C)
    w2 = kn_w2.reshape(S, C, C).transpose(1, 0, 2).reshape(C, S * C)
    wb1 = kn_b1.reshape(F, C)
    wb2 = kn_b2.reshape(C, C)

    return pl.pallas_call(
        _net_kernel,
        out_shape=jax.ShapeDtypeStruct((B, NOUT), f32),
        scratch_shapes=[
            pltpu.VMEM((B * N, S * N + N), f32),
            pltpu.VMEM((B * N, S * C), f32),
            pltpu.VMEM((B * N, C), f32),
            pltpu.VMEM((B * N, C), f32),
            pltpu.VMEM((B * N, C), f32),
        ],
    )(x2, a2, e2, jnp.asarray(_PERM), w1, wb1, root1, bias1.reshape(1, C),
      w2, wb2, root2, bias2.reshape(1, C), dense_w, dense_b.reshape(1, NOUT))


# speedup vs baseline: 68.0203x; 1.1067x over previous
import jax
import jax.numpy as jnp
from jax.experimental import pallas as pl
from jax.experimental.pallas import tpu as pltpu

B, N, F, S, C, NOUT = 32, 32, 32, 4, 32, 16


def _fold_machinery():
    f32 = jnp.float32
    row = jax.lax.broadcasted_iota(jnp.int32, (F, F * C), 0)
    col = jax.lax.broadcasted_iota(jnp.int32, (F, F * C), 1)
    blk_mask = (col // C == row).astype(f32)
    mrow = jax.lax.broadcasted_iota(jnp.int32, (F * C, C), 0)
    mcol = jax.lax.broadcasted_iota(jnp.int32, (F * C, C), 1)
    collapse = (mrow % C == mcol).astype(f32)
    return blk_mask, collapse


def _fold_row(w_row, blk_mask, collapse):
    rep = jnp.broadcast_to(w_row, (F, F * C))
    return jnp.dot(rep * blk_mask, collapse, preferred_element_type=jnp.float32)


def _net_kernel(x2_ref, a2_ref, e2_ref, w1_ref, wb1_ref, root1_ref, b1_ref,
                w2_ref, wb2_ref, root2_ref, b2_ref, dw_ref, db_ref,
                out_ref, ae_s, tu_s, r_s, h_s):
    f32 = jnp.float32
    blk_mask, collapse = _fold_machinery()
    feats = x2_ref[:, :F]

    prow = jax.lax.broadcasted_iota(jnp.int32, (N * S, S * N), 0)
    pcol = jax.lax.broadcasted_iota(jnp.int32, (N * S, S * N), 1)
    perm = ((prow % S) * N + prow // S == pcol).astype(f32)
    et = jnp.dot(e2_ref[:], perm, preferred_element_type=f32)

    a2 = a2_ref[:]
    a4 = jnp.concatenate([a2, a2, a2, a2], axis=1)
    ae_s[:] = jnp.concatenate([a4 * et, a2], axis=1)

    def node_stage(src, w_ref, wb_ref, root_ref, b_ref):
        blocks = [jnp.dot(src, _fold_row(w_ref[s:s + 1, :], blk_mask, collapse),
                          preferred_element_type=f32) for s in range(S)]
        blocks.append(jnp.dot(src, _fold_row(wb_ref[:], blk_mask, collapse),
                              preferred_element_type=f32))
        tu_s[:] = jnp.concatenate(blocks, axis=1)
        r_s[:] = (jnp.dot(src, root_ref[:], preferred_element_type=f32)
                  + jnp.reshape(b_ref[:], (1, C)))

    def conv_rows(b):
        ae = ae_s[b * N:(b + 1) * N, :]
        tb = tu_s[b * N:(b + 1) * N, :]
        tu = jnp.concatenate(
            [tb[:, 0:C], tb[:, C:2 * C], tb[:, 2 * C:3 * C], tb[:, 3 * C:4 * C],
             tb[:, 4 * C:5 * C]], axis=0)
        rb = r_s[b * N:(b + 1) * N, :]
        return jnp.maximum(jnp.dot(ae, tu, preferred_element_type=f32) + rb, 0.0)

    node_stage(feats, w1_ref, wb1_ref, root1_ref, b1_ref)
    for b in range(B):
        h_s[b * N:(b + 1) * N, :] = conv_rows(b)

    node_stage(h_s[:], w2_ref, wb2_ref, root2_ref, b2_ref)

    mcol = (x2_ref[:, F:F + 1] != 0.0).astype(f32)
    rows = []
    for b in range(B):
        h2 = conv_rows(b)
        mb = mcol[b * N:(b + 1) * N, :]
        rows.append(jnp.sum(h2 * mb, axis=0, keepdims=True))
    pooled = jnp.concatenate(rows, axis=0)
    out_ref[:] = (jnp.dot(pooled, dw_ref[:], preferred_element_type=f32)
                  + jnp.reshape(db_ref[:], (1, NOUT)))


def kernel(x, a, e, kn_w1, kn_b1, root1, bias1, kn_w2, kn_b2, root2, bias2,
           dense_w, dense_b):
    f32 = jnp.float32
    x2 = x.reshape(B * N, F + 1)
    a2 = a.reshape(B * N, N)
    e2 = e.reshape(B * N, N * S)
    wb1 = kn_b1.reshape(1, F * C)
    wb2 = kn_b2.reshape(1, C * C)

    return pl.pallas_call(
        _net_kernel,
        out_shape=jax.ShapeDtypeStruct((B, NOUT), f32),
        scratch_shapes=[
            pltpu.VMEM((B * N, S * N + N), f32),
            pltpu.VMEM((B * N, (S + 1) * C), f32),
            pltpu.VMEM((B * N, C), f32),
            pltpu.VMEM((B * N, C), f32),
        ],
    )(x2, a2, e2, kn_w1, wb1, root1, bias1,
      kn_w2, wb2, root2, bias2, dense_w, dense_b)
